# baseline (speedup 1.0000x reference)
import functools

import jax
import jax.numpy as jnp
from jax import lax
from jax.experimental import pallas as pl
from jax.experimental.pallas import tpu as pltpu
from jax.experimental.pallas import tpu_sc as plsc

NC = 2
NS = 16
K = 128
NBUF = 4


def _mesh():
    return plsc.VectorSubcoreMesh(core_axis_name="c", subcore_axis_name="s")


def _untiled():
    return pltpu.CompilerParams(use_tc_tiling_on_sc=False)


@functools.lru_cache(maxsize=None)
def _build_deg_kernel(NCH, N):
    rows = N // NS

    @functools.partial(
        pl.kernel,
        mesh=_mesh(),
        out_type=jax.ShapeDtypeStruct((NC, N, 8), jnp.float32),
        compiler_params=_untiled(),
        scratch_types=[
            pltpu.VMEM((NCH, K), jnp.int32),
            pltpu.VMEM((K, 8), jnp.float32),
            pltpu.VMEM_SHARED((N, 8), jnp.float32),
            pltpu.SemaphoreType.DMA,
        ],
    )
    def deg_kernel(dst_hbm, ones_hbm, zeros_hbm, out_hbm, didx, ones_v, acc_sh, sem):
        cid = lax.axis_index("c")
        sid = lax.axis_index("s")
        w = cid * NS + sid
        pltpu.sync_copy(zeros_hbm.at[pl.ds(sid * rows, rows)],
                        acc_sh.at[pl.ds(sid * rows, rows)])
        pltpu.sync_copy(dst_hbm.at[w], didx)
        pltpu.sync_copy(ones_hbm, ones_v)
        plsc.subcore_barrier()

        def fire(j, carry):
            pltpu.async_copy(ones_v, acc_sh.at[didx.at[j]], sem, add=True)
            return carry

        lax.fori_loop(0, NCH, fire, 0)

        def drain(j, carry):
            pltpu.make_async_copy(ones_v, acc_sh.at[didx.at[j]], sem).wait()
            return carry

        lax.fori_loop(0, NCH, drain, 0)
        plsc.subcore_barrier()
        pltpu.sync_copy(acc_sh.at[pl.ds(sid * rows, rows)],
                        out_hbm.at[cid, pl.ds(sid * rows, rows)])

    return deg_kernel


@functools.lru_cache(maxsize=None)
def _build_scatter_kernel(NCH, N, D):
    if D >= 32:
        return _build_scatter_sync(NCH, N, D)
    return _build_scatter_ring(NCH, N, D)


def _build_scatter_sync(NCH, N, D):
    rows = N // NS
    assert NCH % 2 == 0
    half = NCH // 2 - 1

    @functools.partial(
        pl.kernel,
        mesh=_mesh(),
        out_type=jax.ShapeDtypeStruct((NC, N, D), jnp.float32),
        compiler_params=_untiled(),
        scratch_types=[
            pltpu.VMEM((NCH, K), jnp.int32),
            pltpu.VMEM((NCH, K), jnp.int32),
            pltpu.VMEM((K, D), jnp.float32),
            pltpu.VMEM((K, D), jnp.float32),
            pltpu.SemaphoreType.DMA,
            pltpu.SemaphoreType.DMA,
            pltpu.VMEM_SHARED((N, D), jnp.float32),
        ],
    )
    def scat_kernel(src_hbm, dst_hbm, g_hbm, zeros_hbm, out_hbm,
                    sidx, didx, buf0, buf1, sem0, sem1, acc_sh):
        cid = lax.axis_index("c")
        sid = lax.axis_index("s")
        w = cid * NS + sid
        pltpu.sync_copy(zeros_hbm.at[pl.ds(sid * rows, rows)],
                        acc_sh.at[pl.ds(sid * rows, rows)])
        pltpu.sync_copy(src_hbm.at[w], sidx)
        pltpu.sync_copy(dst_hbm.at[w], didx)
        plsc.subcore_barrier()

        pltpu.async_copy(g_hbm.at[sidx.at[0]], buf0, sem0)

        def body(j, carry):
            c0 = 2 * j
            c1 = c0 + 1
            pltpu.make_async_copy(g_hbm.at[sidx.at[c0]], buf0, sem0).wait()
            pltpu.async_copy(g_hbm.at[sidx.at[c1]], buf1, sem1)
            pltpu.sync_copy(buf0, acc_sh.at[didx.at[c0]], add=True)
            pltpu.make_async_copy(g_hbm.at[sidx.at[c1]], buf1, sem1).wait()
            pltpu.async_copy(g_hbm.at[sidx.at[c1 + 1]], buf0, sem0)
            pltpu.sync_copy(buf1, acc_sh.at[didx.at[c1]], add=True)
            return carry

        lax.fori_loop(0, half, body, 0)
        pltpu.make_async_copy(g_hbm.at[sidx.at[NCH - 2]], buf0, sem0).wait()
        pltpu.async_copy(g_hbm.at[sidx.at[NCH - 1]], buf1, sem1)
        pltpu.sync_copy(buf0, acc_sh.at[didx.at[NCH - 2]], add=True)
        pltpu.make_async_copy(g_hbm.at[sidx.at[NCH - 1]], buf1, sem1).wait()
        pltpu.sync_copy(buf1, acc_sh.at[didx.at[NCH - 1]], add=True)

        plsc.subcore_barrier()
        pltpu.sync_copy(acc_sh.at[pl.ds(sid * rows, rows)],
                        out_hbm.at[cid, pl.ds(sid * rows, rows)])

    return scat_kernel


def _build_scatter_ring(NCH, N, D):
    rows = N // NS
    assert NCH % NBUF == 0

    @functools.partial(
        pl.kernel,
        mesh=_mesh(),
        out_type=jax.ShapeDtypeStruct((NC, N, D), jnp.float32),
        compiler_params=_untiled(),
        scratch_types=[
            pltpu.VMEM((NCH, K), jnp.int32),
            pltpu.VMEM((NCH, K), jnp.int32),
            [pltpu.VMEM((K, D), jnp.float32) for _ in range(NBUF)],
            [pltpu.SemaphoreType.DMA for _ in range(NBUF)],
            [pltpu.SemaphoreType.DMA for _ in range(NBUF)],
            pltpu.VMEM_SHARED((N, D), jnp.float32),
        ],
    )
    def scat_kernel(src_hbm, dst_hbm, g_hbm, zeros_hbm, out_hbm,
                    sidx, didx, bufs, gsems, ssems, acc_sh):
        cid = lax.axis_index("c")
        sid = lax.axis_index("s")
        w = cid * NS + sid
        pltpu.sync_copy(zeros_hbm.at[pl.ds(sid * rows, rows)],
                        acc_sh.at[pl.ds(sid * rows, rows)])
        pltpu.sync_copy(src_hbm.at[w], sidx)
        pltpu.sync_copy(dst_hbm.at[w], didx)
        plsc.subcore_barrier()

        for b in range(NBUF - 1):
            pltpu.async_copy(g_hbm.at[sidx.at[b]], bufs[b], gsems[b])

        def body(jj, carry):
            for b in range(NBUF):
                c = NBUF * jj + b
                pltpu.make_async_copy(g_hbm.at[sidx.at[c]], bufs[b], gsems[b]).wait()
                pltpu.async_copy(bufs[b], acc_sh.at[didx.at[c]], ssems[b], add=True)
                nb = (b + NBUF - 1) % NBUF

                @pl.when(c + NBUF - 1 < NCH)
                def _():
                    @pl.when(c >= 1)
                    def _():
                        pltpu.make_async_copy(
                            bufs[nb], acc_sh.at[didx.at[jnp.maximum(c - 1, 0)]],
                            ssems[nb]).wait()
                    pltpu.async_copy(g_hbm.at[sidx.at[c + NBUF - 1]], bufs[nb],
                                     gsems[nb])
            return carry

        lax.fori_loop(0, NCH // NBUF, body, 0)
        for b in range(NBUF):
            c = NCH - NBUF + b
            pltpu.make_async_copy(bufs[c % NBUF], acc_sh.at[didx.at[c]],
                                  ssems[c % NBUF]).wait()

        plsc.subcore_barrier()
        pltpu.sync_copy(acc_sh.at[pl.ds(sid * rows, rows)],
                        out_hbm.at[cid, pl.ds(sid * rows, rows)])

    return scat_kernel


def _dis(parts):
    deg = parts[0, :, 0:1] + parts[1, :, 0:1] + 1.0
    return lax.rsqrt(deg)


def _tc1_body(parts_ref, x_ref, w1_ref, g1_ref):
    dis = _dis(parts_ref[...])
    h = jnp.dot(x_ref[...], w1_ref[...], preferred_element_type=jnp.float32,
                precision=lax.Precision.HIGHEST)
    g1_ref[...] = dis * h


def _tc2_body(parts_ref, s_ref, g1_ref, b1_ref, w2_ref, g2_ref):
    dis = _dis(parts_ref[...])
    s = s_ref[0] + s_ref[1] + g1_ref[...]
    h = jnp.maximum(dis * s + b1_ref[...], 0.0)
    g2_ref[...] = dis * jnp.dot(h, w2_ref[...], preferred_element_type=jnp.float32,
                                precision=lax.Precision.HIGHEST)


def _tc3_body(parts_ref, s_ref, g2_ref, b2_ref, out_ref):
    dis = _dis(parts_ref[...])
    out_ref[...] = dis * (s_ref[0] + s_ref[1] + g2_ref[...]) + b2_ref[...]


def kernel(x, edge_index, W1, b1, W2, b2):
    N, F = x.shape
    E = edge_index.shape[1]
    H = W1.shape[1]
    C = W2.shape[1]
    Dp = 8
    NP = ((N + 127) // 128) * 128
    W = NC * NS
    epw = E // W
    NCH = -(-epw // K)
    NCH = -(-NCH // NBUF) * NBUF
    pad = NCH * K - epw

    src3 = jnp.pad(edge_index[0].reshape(W, epw), ((0, 0), (0, pad)),
                   constant_values=0).reshape(W, NCH, K)
    padv = N + (jnp.arange(pad, dtype=jnp.int32) % (NP - N))
    dst3 = jnp.concatenate(
        [edge_index[1].reshape(W, epw),
         jnp.broadcast_to(padv, (W, pad))], axis=1).reshape(W, NCH, K)

    xp = jnp.pad(x, ((0, NP - N), (0, 0)))
    ones8 = jnp.ones((K, 8), jnp.float32)
    zeros8 = jnp.zeros((NP, Dp), jnp.float32)
    zerosH = jnp.zeros((NP, H), jnp.float32)
    W2p = jnp.zeros((H, Dp), jnp.float32).at[:, :C].set(W2)
    b2p = jnp.zeros((1, Dp), jnp.float32).at[0, :C].set(b2)
    b1r = b1.reshape(1, H)

    parts = _build_deg_kernel(NCH, NP)(dst3, ones8, zeros8)
    g1 = pl.pallas_call(
        _tc1_body, out_shape=jax.ShapeDtypeStruct((NP, H), jnp.float32),
    )(parts, xp, W1)
    s1 = _build_scatter_kernel(NCH, NP, H)(src3, dst3, g1, zerosH)
    g2 = pl.pallas_call(
        _tc2_body, out_shape=jax.ShapeDtypeStruct((NP, Dp), jnp.float32),
    )(parts, s1, g1, b1r, W2p)
    s2 = _build_scatter_kernel(NCH, NP, Dp)(src3, dst3, g2, zeros8)
    out = pl.pallas_call(
        _tc3_body, out_shape=jax.ShapeDtypeStruct((NP, Dp), jnp.float32),
    )(parts, s2, g2, b2p)
    return out[:N, :C]


# speedup vs baseline: 1.1522x; 1.1522x over previous
import functools

import jax
import jax.numpy as jnp
from jax import lax
from jax.experimental import pallas as pl
from jax.experimental.pallas import tpu as pltpu
from jax.experimental.pallas import tpu_sc as plsc

NC = 2
NS = 16
K = 128


def _mesh():
    return plsc.VectorSubcoreMesh(core_axis_name="c", subcore_axis_name="s")


def _untiled():
    return pltpu.CompilerParams(use_tc_tiling_on_sc=False)


@functools.lru_cache(maxsize=None)
def _build_deg_kernel(NCH, N):
    rows = N // NS

    @functools.partial(
        pl.kernel,
        mesh=_mesh(),
        out_type=jax.ShapeDtypeStruct((NC, N, 8), jnp.float32),
        compiler_params=_untiled(),
        scratch_types=[
            pltpu.VMEM((NCH, K), jnp.int32),
            pltpu.VMEM((K, 8), jnp.float32),
            pltpu.VMEM_SHARED((N, 8), jnp.float32),
        ],
    )
    def deg_kernel(dst_hbm, ones_hbm, zeros_hbm, out_hbm, didx, ones_v, acc_sh):
        cid = lax.axis_index("c")
        sid = lax.axis_index("s")
        w = cid * NS + sid
        pltpu.sync_copy(zeros_hbm.at[pl.ds(sid * rows, rows)],
                        acc_sh.at[pl.ds(sid * rows, rows)])
        pltpu.sync_copy(dst_hbm.at[w], didx)
        pltpu.sync_copy(ones_hbm, ones_v)
        plsc.subcore_barrier()

        def body(j, carry):
            pltpu.sync_copy(ones_v, acc_sh.at[didx.at[j]], add=True)
            return carry

        lax.fori_loop(0, NCH, body, 0)
        plsc.subcore_barrier()
        pltpu.sync_copy(acc_sh.at[pl.ds(sid * rows, rows)],
                        out_hbm.at[cid, pl.ds(sid * rows, rows)])

    return deg_kernel


@functools.lru_cache(maxsize=None)
def _build_scatter_kernel(NCH, N, D):
    rows = N // NS
    half = (NCH - 1) // 2

    @functools.partial(
        pl.kernel,
        mesh=_mesh(),
        out_type=jax.ShapeDtypeStruct((NC, N, D), jnp.float32),
        compiler_params=_untiled(),
        scratch_types=[
            pltpu.VMEM((NCH, K), jnp.int32),
            pltpu.VMEM((NCH, K), jnp.int32),
            pltpu.VMEM((K, D), jnp.float32),
            pltpu.VMEM((K, D), jnp.float32),
            pltpu.SemaphoreType.DMA,
            pltpu.SemaphoreType.DMA,
            pltpu.VMEM_SHARED((N, D), jnp.float32),
        ],
    )
    def scat_kernel(src_hbm, dst_hbm, g_hbm, zeros_hbm, out_hbm,
                    sidx, didx, buf0, buf1, sem0, sem1, acc_sh):
        cid = lax.axis_index("c")
        sid = lax.axis_index("s")
        w = cid * NS + sid
        pltpu.sync_copy(zeros_hbm.at[pl.ds(sid * rows, rows)],
                        acc_sh.at[pl.ds(sid * rows, rows)])
        pltpu.sync_copy(src_hbm.at[w], sidx)
        pltpu.sync_copy(dst_hbm.at[w], didx)
        plsc.subcore_barrier()

        pltpu.async_copy(g_hbm.at[sidx.at[0]], buf0, sem0)

        def body(j, carry):
            c0 = 2 * j
            c1 = c0 + 1
            pltpu.make_async_copy(g_hbm.at[sidx.at[c0]], buf0, sem0).wait()
            pltpu.async_copy(g_hbm.at[sidx.at[c1]], buf1, sem1)
            pltpu.sync_copy(buf0, acc_sh.at[didx.at[c0]], add=True)
            pltpu.make_async_copy(g_hbm.at[sidx.at[c1]], buf1, sem1).wait()
            pltpu.async_copy(g_hbm.at[sidx.at[c1 + 1]], buf0, sem0)
            pltpu.sync_copy(buf1, acc_sh.at[didx.at[c1]], add=True)
            return carry

        lax.fori_loop(0, half, body, 0)
        pltpu.make_async_copy(g_hbm.at[sidx.at[NCH - 1]], buf0, sem0).wait()
        pltpu.sync_copy(buf0, acc_sh.at[didx.at[NCH - 1]], add=True)

        plsc.subcore_barrier()
        pltpu.sync_copy(acc_sh.at[pl.ds(sid * rows, rows)],
                        out_hbm.at[cid, pl.ds(sid * rows, rows)])

    return scat_kernel


def _dis(parts):
    deg = parts[0, :, 0:1] + parts[1, :, 0:1] + 1.0
    return lax.rsqrt(deg)


def _tc1_body(parts_ref, x_ref, w1_ref, g1_ref):
    dis = _dis(parts_ref[...])
    h = jnp.dot(x_ref[...], w1_ref[...], preferred_element_type=jnp.float32,
                precision=lax.Precision.HIGHEST)
    g1_ref[...] = dis * h


def _tc2_body(parts_ref, s_ref, g1_ref, b1_ref, w2_ref, g2_ref):
    dis = _dis(parts_ref[...])
    s = s_ref[0] + s_ref[1] + g1_ref[...]
    h = jnp.maximum(dis * s + b1_ref[...], 0.0)
    g2_ref[...] = dis * jnp.dot(h, w2_ref[...], preferred_element_type=jnp.float32,
                                precision=lax.Precision.HIGHEST)


def _tc3_body(parts_ref, s_ref, g2_ref, b2_ref, out_ref):
    dis = _dis(parts_ref[...])
    out_ref[...] = dis * (s_ref[0] + s_ref[1] + g2_ref[...]) + b2_ref[...]


def kernel(x, edge_index, W1, b1, W2, b2):
    N, F = x.shape
    E = edge_index.shape[1]
    H = W1.shape[1]
    C = W2.shape[1]
    Dp = 8
    NP = ((N + 127) // 128) * 128
    W = NC * NS
    epw = E // W
    NCH = (epw + K - 1) // K
    pad = NCH * K - epw

    src3 = jnp.pad(edge_index[0].reshape(W, epw), ((0, 0), (0, pad)),
                   constant_values=0).reshape(W, NCH, K)
    dst3 = jnp.pad(edge_index[1].reshape(W, epw), ((0, 0), (0, pad)),
                   constant_values=N).reshape(W, NCH, K)

    xp = jnp.pad(x, ((0, NP - N), (0, 0)))
    ones8 = jnp.ones((K, 8), jnp.float32)
    zeros8 = jnp.zeros((NP, Dp), jnp.float32)
    zerosH = jnp.zeros((NP, H), jnp.float32)
    W2p = jnp.zeros((H, Dp), jnp.float32).at[:, :C].set(W2)
    b2p = jnp.zeros((1, Dp), jnp.float32).at[0, :C].set(b2)
    b1r = b1.reshape(1, H)

    parts = _build_deg_kernel(NCH, NP)(dst3, ones8, zeros8)
    g1 = pl.pallas_call(
        _tc1_body, out_shape=jax.ShapeDtypeStruct((NP, H), jnp.float32),
    )(parts, xp, W1)
    s1 = _build_scatter_kernel(NCH, NP, H)(src3, dst3, g1, zerosH)
    g2 = pl.pallas_call(
        _tc2_body, out_shape=jax.ShapeDtypeStruct((NP, Dp), jnp.float32),
    )(parts, s1, g1, b1r, W2p)
    s2 = _build_scatter_kernel(NCH, NP, Dp)(src3, dst3, g2, zeros8)
    out = pl.pallas_call(
        _tc3_body, out_shape=jax.ShapeDtypeStruct((NP, Dp), jnp.float32),
    )(parts, s2, g2, b2p)
    return out[:N, :C]


# speedup vs baseline: 1.2086x; 1.0489x over previous
import functools

import jax
import jax.numpy as jnp
from jax import lax
from jax.experimental import pallas as pl
from jax.experimental.pallas import tpu as pltpu
from jax.experimental.pallas import tpu_sc as plsc

NC = 2
NS = 16
K = 128
NBUF = 4


def _mesh():
    return plsc.VectorSubcoreMesh(core_axis_name="c", subcore_axis_name="s")


def _untiled():
    return pltpu.CompilerParams(use_tc_tiling_on_sc=False)


@functools.lru_cache(maxsize=None)
def _build_deg_kernel(NCH, N):
    rows = N // NS

    @functools.partial(
        pl.kernel,
        mesh=_mesh(),
        out_type=jax.ShapeDtypeStruct((NC, N, 8), jnp.float32),
        compiler_params=_untiled(),
        scratch_types=[
            pltpu.VMEM((NCH, K), jnp.int32),
            pltpu.VMEM((K, 8), jnp.float32),
            pltpu.VMEM_SHARED((N, 8), jnp.float32),
        ],
    )
    def deg_kernel(dst_hbm, ones_hbm, zeros_hbm, out_hbm, didx, ones_v, acc_sh):
        cid = lax.axis_index("c")
        sid = lax.axis_index("s")
        w = cid * NS + sid
        pltpu.sync_copy(zeros_hbm.at[pl.ds(sid * rows, rows)],
                        acc_sh.at[pl.ds(sid * rows, rows)])
        pltpu.sync_copy(dst_hbm.at[w], didx)
        pltpu.sync_copy(ones_hbm, ones_v)
        plsc.subcore_barrier()

        def body(j, carry):
            pltpu.sync_copy(ones_v, acc_sh.at[didx.at[j]], add=True)
            return carry

        lax.fori_loop(0, NCH, body, 0)
        plsc.subcore_barrier()
        pltpu.sync_copy(acc_sh.at[pl.ds(sid * rows, rows)],
                        out_hbm.at[cid, pl.ds(sid * rows, rows)])

    return deg_kernel


@functools.lru_cache(maxsize=None)
def _build_scatter_kernel(NCH, N, D):
    rows = N // NS
    half = (NCH - 1) // 2

    @functools.partial(
        pl.kernel,
        mesh=_mesh(),
        out_type=jax.ShapeDtypeStruct((NC, N, D), jnp.float32),
        compiler_params=_untiled(),
        scratch_types=[
            pltpu.VMEM((NCH, K), jnp.int32),
            pltpu.VMEM((NCH, K), jnp.int32),
            pltpu.VMEM((K, D), jnp.float32),
            pltpu.VMEM((K, D), jnp.float32),
            pltpu.SemaphoreType.DMA,
            pltpu.SemaphoreType.DMA,
            pltpu.VMEM_SHARED((N, D), jnp.float32),
        ],
    )
    def scat_kernel(src_hbm, dst_hbm, g_hbm, zeros_hbm, out_hbm,
                    sidx, didx, buf0, buf1, sem0, sem1, acc_sh):
        cid = lax.axis_index("c")
        sid = lax.axis_index("s")
        w = cid * NS + sid
        pltpu.sync_copy(zeros_hbm.at[pl.ds(sid * rows, rows)],
                        acc_sh.at[pl.ds(sid * rows, rows)])
        pltpu.sync_copy(src_hbm.at[w], sidx)
        pltpu.sync_copy(dst_hbm.at[w], didx)
        plsc.subcore_barrier()

        pltpu.async_copy(g_hbm.at[sidx.at[0]], buf0, sem0)

        def body(j, carry):
            c0 = 2 * j
            c1 = c0 + 1
            pltpu.make_async_copy(g_hbm.at[sidx.at[c0]], buf0, sem0).wait()
            pltpu.async_copy(g_hbm.at[sidx.at[c1]], buf1, sem1)
            pltpu.sync_copy(buf0, acc_sh.at[didx.at[c0]], add=True)
            pltpu.make_async_copy(g_hbm.at[sidx.at[c1]], buf1, sem1).wait()
            pltpu.async_copy(g_hbm.at[sidx.at[c1 + 1]], buf0, sem0)
            pltpu.sync_copy(buf1, acc_sh.at[didx.at[c1]], add=True)
            return carry

        lax.fori_loop(0, half, body, 0)
        pltpu.make_async_copy(g_hbm.at[sidx.at[NCH - 1]], buf0, sem0).wait()
        pltpu.sync_copy(buf0, acc_sh.at[didx.at[NCH - 1]], add=True)

        plsc.subcore_barrier()
        pltpu.sync_copy(acc_sh.at[pl.ds(sid * rows, rows)],
                        out_hbm.at[cid, pl.ds(sid * rows, rows)])

    return scat_kernel


def _build_scatter_ring(NCH, N, D):
    rows = N // NS
    assert NCH % NBUF == 0

    @functools.partial(
        pl.kernel,
        mesh=_mesh(),
        out_type=jax.ShapeDtypeStruct((NC, N, D), jnp.float32),
        compiler_params=_untiled(),
        scratch_types=[
            pltpu.VMEM((NCH, K), jnp.int32),
            pltpu.VMEM((NCH, K), jnp.int32),
            [pltpu.VMEM((K, D), jnp.float32) for _ in range(NBUF)],
            [pltpu.SemaphoreType.DMA for _ in range(NBUF)],
            [pltpu.SemaphoreType.DMA for _ in range(NBUF)],
            pltpu.VMEM_SHARED((N, D), jnp.float32),
        ],
    )
    def scat_kernel(src_hbm, dst_hbm, g_hbm, zeros_hbm, out_hbm,
                    sidx, didx, bufs, gsems, ssems, acc_sh):
        cid = lax.axis_index("c")
        sid = lax.axis_index("s")
        w = cid * NS + sid
        pltpu.sync_copy(zeros_hbm.at[pl.ds(sid * rows, rows)],
                        acc_sh.at[pl.ds(sid * rows, rows)])
        pltpu.sync_copy(src_hbm.at[w], sidx)
        pltpu.sync_copy(dst_hbm.at[w], didx)
        plsc.subcore_barrier()

        for b in range(NBUF - 1):
            pltpu.async_copy(g_hbm.at[sidx.at[b]], bufs[b], gsems[b])

        def body(jj, carry):
            for b in range(NBUF):
                c = NBUF * jj + b
                pltpu.make_async_copy(g_hbm.at[sidx.at[c]], bufs[b], gsems[b]).wait()
                pltpu.async_copy(bufs[b], acc_sh.at[didx.at[c]], ssems[b], add=True)
                nb = (b + NBUF - 1) % NBUF

                @pl.when(c + NBUF - 1 < NCH)
                def _():
                    @pl.when(c >= 1)
                    def _():
                        pltpu.make_async_copy(
                            bufs[nb], acc_sh.at[didx.at[jnp.maximum(c - 1, 0)]],
                            ssems[nb]).wait()
                    pltpu.async_copy(g_hbm.at[sidx.at[c + NBUF - 1]], bufs[nb],
                                     gsems[nb])
            return carry

        lax.fori_loop(0, NCH // NBUF, body, 0)
        for b in range(NBUF):
            c = NCH - NBUF + b
            pltpu.make_async_copy(bufs[c % NBUF], acc_sh.at[didx.at[c]],
                                  ssems[c % NBUF]).wait()

        plsc.subcore_barrier()
        pltpu.sync_copy(acc_sh.at[pl.ds(sid * rows, rows)],
                        out_hbm.at[cid, pl.ds(sid * rows, rows)])

    return scat_kernel


def _dis(parts):
    deg = parts[0, :, 0:1] + parts[1, :, 0:1] + 1.0
    return lax.rsqrt(deg)


def _tc1_body(parts_ref, x_ref, w1_ref, g1_ref):
    dis = _dis(parts_ref[...])
    h = jnp.dot(x_ref[...], w1_ref[...], preferred_element_type=jnp.float32,
                precision=lax.Precision.HIGHEST)
    g1_ref[...] = dis * h


def _tc2_body(parts_ref, s_ref, g1_ref, b1_ref, w2_ref, g2_ref):
    dis = _dis(parts_ref[...])
    s = s_ref[0] + s_ref[1] + g1_ref[...]
    h = jnp.maximum(dis * s + b1_ref[...], 0.0)
    g2_ref[...] = dis * jnp.dot(h, w2_ref[...], preferred_element_type=jnp.float32,
                                precision=lax.Precision.HIGHEST)


def _tc3_body(parts_ref, s_ref, g2_ref, b2_ref, out_ref):
    dis = _dis(parts_ref[...])
    out_ref[...] = dis * (s_ref[0] + s_ref[1] + g2_ref[...]) + b2_ref[...]


def kernel(x, edge_index, W1, b1, W2, b2):
    N, F = x.shape
    E = edge_index.shape[1]
    H = W1.shape[1]
    C = W2.shape[1]
    Dp = 8
    NP = ((N + 127) // 128) * 128
    W = NC * NS
    epw = E // W
    NCH = (epw + K - 1) // K
    pad = NCH * K - epw

    src3 = jnp.pad(edge_index[0].reshape(W, epw), ((0, 0), (0, pad)),
                   constant_values=0).reshape(W, NCH, K)
    dst3 = jnp.pad(edge_index[1].reshape(W, epw), ((0, 0), (0, pad)),
                   constant_values=N).reshape(W, NCH, K)
    NCH2 = -(-NCH // NBUF) * NBUF
    pad2 = NCH2 * K - epw
    src3b = jnp.pad(edge_index[0].reshape(W, epw), ((0, 0), (0, pad2)),
                    constant_values=0).reshape(W, NCH2, K)
    padv = N + (jnp.arange(pad2, dtype=jnp.int32) % (NP - N))
    dst3b = jnp.concatenate(
        [edge_index[1].reshape(W, epw),
         jnp.broadcast_to(padv, (W, pad2))], axis=1).reshape(W, NCH2, K)

    xp = jnp.pad(x, ((0, NP - N), (0, 0)))
    ones8 = jnp.ones((K, 8), jnp.float32)
    zeros8 = jnp.zeros((NP, Dp), jnp.float32)
    zerosH = jnp.zeros((NP, H), jnp.float32)
    W2p = jnp.zeros((H, Dp), jnp.float32).at[:, :C].set(W2)
    b2p = jnp.zeros((1, Dp), jnp.float32).at[0, :C].set(b2)
    b1r = b1.reshape(1, H)

    parts = _build_deg_kernel(NCH, NP)(dst3, ones8, zeros8)
    g1 = pl.pallas_call(
        _tc1_body, out_shape=jax.ShapeDtypeStruct((NP, H), jnp.float32),
    )(parts, xp, W1)
    s1 = _build_scatter_kernel(NCH, NP, H)(src3, dst3, g1, zerosH)
    g2 = pl.pallas_call(
        _tc2_body, out_shape=jax.ShapeDtypeStruct((NP, Dp), jnp.float32),
    )(parts, s1, g1, b1r, W2p)
    s2 = _build_scatter_ring(NCH2, NP, Dp)(src3b, dst3b, g2, zeros8)
    out = pl.pallas_call(
        _tc3_body, out_shape=jax.ShapeDtypeStruct((NP, Dp), jnp.float32),
    )(parts, s2, g2, b2p)
    return out[:N, :C]


# speedup vs baseline: 1.2250x; 1.0136x over previous
import functools

import jax
import jax.numpy as jnp
from jax import lax
from jax.experimental import pallas as pl
from jax.experimental.pallas import tpu as pltpu
from jax.experimental.pallas import tpu_sc as plsc

NC = 2
NS = 16
K = 128
NBUF = 4


def _mesh():
    return plsc.VectorSubcoreMesh(core_axis_name="c", subcore_axis_name="s")


def _untiled():
    return pltpu.CompilerParams(use_tc_tiling_on_sc=False)


@functools.lru_cache(maxsize=None)
def _build_deg_kernel(NCH, N):
    rows = N // NS

    @functools.partial(
        pl.kernel,
        mesh=_mesh(),
        out_type=jax.ShapeDtypeStruct((NC, N, 8), jnp.float32),
        compiler_params=_untiled(),
        scratch_types=[
            pltpu.VMEM((NCH, K), jnp.int32),
            pltpu.VMEM((K, 8), jnp.float32),
            pltpu.VMEM_SHARED((N, 8), jnp.float32),
            pltpu.SemaphoreType.DMA,
        ],
    )
    def deg_kernel(dst_hbm, ones_hbm, zeros_hbm, out_hbm, didx, ones_v, acc_sh, sem):
        cid = lax.axis_index("c")
        sid = lax.axis_index("s")
        w = cid * NS + sid
        pltpu.sync_copy(zeros_hbm.at[pl.ds(sid * rows, rows)],
                        acc_sh.at[pl.ds(sid * rows, rows)])
        pltpu.sync_copy(dst_hbm.at[w], didx)
        pltpu.sync_copy(ones_hbm, ones_v)
        plsc.subcore_barrier()

        def fire(j, carry):
            pltpu.async_copy(ones_v, acc_sh.at[didx.at[j]], sem, add=True)
            return carry

        lax.fori_loop(0, NCH, fire, 0)

        def drain(j, carry):
            pltpu.make_async_copy(ones_v, acc_sh.at[didx.at[j]], sem).wait()
            return carry

        lax.fori_loop(0, NCH, drain, 0)
        plsc.subcore_barrier()
        pltpu.sync_copy(acc_sh.at[pl.ds(sid * rows, rows)],
                        out_hbm.at[cid, pl.ds(sid * rows, rows)])

    return deg_kernel


@functools.lru_cache(maxsize=None)
def _build_scatter_kernel(NCH, N, D):
    rows = N // NS
    half = (NCH - 1) // 2

    @functools.partial(
        pl.kernel,
        mesh=_mesh(),
        out_type=jax.ShapeDtypeStruct((NC, N, D), jnp.float32),
        compiler_params=_untiled(),
        scratch_types=[
            pltpu.VMEM((NCH, K), jnp.int32),
            pltpu.VMEM((NCH, K), jnp.int32),
            pltpu.VMEM((K, D), jnp.float32),
            pltpu.VMEM((K, D), jnp.float32),
            pltpu.SemaphoreType.DMA,
            pltpu.SemaphoreType.DMA,
            pltpu.VMEM_SHARED((N, D), jnp.float32),
        ],
    )
    def scat_kernel(src_hbm, dst_hbm, g_hbm, zeros_hbm, out_hbm,
                    sidx, didx, buf0, buf1, sem0, sem1, acc_sh):
        cid = lax.axis_index("c")
        sid = lax.axis_index("s")
        w = cid * NS + sid
        pltpu.sync_copy(zeros_hbm.at[pl.ds(sid * rows, rows)],
                        acc_sh.at[pl.ds(sid * rows, rows)])
        pltpu.sync_copy(src_hbm.at[w], sidx)
        pltpu.sync_copy(dst_hbm.at[w], didx)
        plsc.subcore_barrier()

        pltpu.async_copy(g_hbm.at[sidx.at[0]], buf0, sem0)

        def body(j, carry):
            c0 = 2 * j
            c1 = c0 + 1
            pltpu.make_async_copy(g_hbm.at[sidx.at[c0]], buf0, sem0).wait()
            pltpu.async_copy(g_hbm.at[sidx.at[c1]], buf1, sem1)
            pltpu.sync_copy(buf0, acc_sh.at[didx.at[c0]], add=True)
            pltpu.make_async_copy(g_hbm.at[sidx.at[c1]], buf1, sem1).wait()
            pltpu.async_copy(g_hbm.at[sidx.at[c1 + 1]], buf0, sem0)
            pltpu.sync_copy(buf1, acc_sh.at[didx.at[c1]], add=True)
            return carry

        lax.fori_loop(0, half, body, 0)
        pltpu.make_async_copy(g_hbm.at[sidx.at[NCH - 1]], buf0, sem0).wait()
        pltpu.sync_copy(buf0, acc_sh.at[didx.at[NCH - 1]], add=True)

        plsc.subcore_barrier()
        pltpu.sync_copy(acc_sh.at[pl.ds(sid * rows, rows)],
                        out_hbm.at[cid, pl.ds(sid * rows, rows)])

    return scat_kernel


def _build_scatter_ring(NCH, N, D):
    rows = N // NS
    assert NCH % NBUF == 0

    @functools.partial(
        pl.kernel,
        mesh=_mesh(),
        out_type=jax.ShapeDtypeStruct((NC, N, D), jnp.float32),
        compiler_params=_untiled(),
        scratch_types=[
            pltpu.VMEM((NCH, K), jnp.int32),
            pltpu.VMEM((NCH, K), jnp.int32),
            [pltpu.VMEM((K, D), jnp.float32) for _ in range(NBUF)],
            [pltpu.SemaphoreType.DMA for _ in range(NBUF)],
            [pltpu.SemaphoreType.DMA for _ in range(NBUF)],
            pltpu.VMEM_SHARED((N, D), jnp.float32),
        ],
    )
    def scat_kernel(src_hbm, dst_hbm, g_hbm, zeros_hbm, out_hbm,
                    sidx, didx, bufs, gsems, ssems, acc_sh):
        cid = lax.axis_index("c")
        sid = lax.axis_index("s")
        w = cid * NS + sid
        pltpu.sync_copy(zeros_hbm.at[pl.ds(sid * rows, rows)],
                        acc_sh.at[pl.ds(sid * rows, rows)])
        pltpu.sync_copy(src_hbm.at[w], sidx)
        pltpu.sync_copy(dst_hbm.at[w], didx)
        plsc.subcore_barrier()

        for b in range(NBUF - 1):
            pltpu.async_copy(g_hbm.at[sidx.at[b]], bufs[b], gsems[b])

        def body(jj, carry):
            for b in range(NBUF):
                c = NBUF * jj + b
                pltpu.make_async_copy(g_hbm.at[sidx.at[c]], bufs[b], gsems[b]).wait()
                pltpu.async_copy(bufs[b], acc_sh.at[didx.at[c]], ssems[b], add=True)
                nb = (b + NBUF - 1) % NBUF

                @pl.when(c + NBUF - 1 < NCH)
                def _():
                    @pl.when(c >= 1)
                    def _():
                        pltpu.make_async_copy(
                            bufs[nb], acc_sh.at[didx.at[jnp.maximum(c - 1, 0)]],
                            ssems[nb]).wait()
                    pltpu.async_copy(g_hbm.at[sidx.at[c + NBUF - 1]], bufs[nb],
                                     gsems[nb])
            return carry

        lax.fori_loop(0, NCH // NBUF, body, 0)
        for b in range(NBUF):
            c = NCH - NBUF + b
            pltpu.make_async_copy(bufs[c % NBUF], acc_sh.at[didx.at[c]],
                                  ssems[c % NBUF]).wait()

        plsc.subcore_barrier()
        pltpu.sync_copy(acc_sh.at[pl.ds(sid * rows, rows)],
                        out_hbm.at[cid, pl.ds(sid * rows, rows)])

    return scat_kernel


def _dis(parts):
    deg = parts[0, :, 0:1] + parts[1, :, 0:1] + 1.0
    return lax.rsqrt(deg)


def _tc1_body(parts_ref, x_ref, w1_ref, g1_ref):
    dis = _dis(parts_ref[...])
    h = jnp.dot(x_ref[...], w1_ref[...], preferred_element_type=jnp.float32,
                precision=lax.Precision.HIGHEST)
    g1_ref[...] = dis * h


def _tc2_body(parts_ref, s_ref, g1_ref, b1_ref, w2_ref, g2_ref):
    dis = _dis(parts_ref[...])
    s = s_ref[0] + s_ref[1] + g1_ref[...]
    h = jnp.maximum(dis * s + b1_ref[...], 0.0)
    g2_ref[...] = dis * jnp.dot(h, w2_ref[...], preferred_element_type=jnp.float32,
                                precision=lax.Precision.HIGHEST)


def _tc3_body(parts_ref, s_ref, g2_ref, b2_ref, out_ref):
    dis = _dis(parts_ref[...])
    out_ref[...] = dis * (s_ref[0] + s_ref[1] + g2_ref[...]) + b2_ref[...]


def kernel(x, edge_index, W1, b1, W2, b2):
    N, F = x.shape
    E = edge_index.shape[1]
    H = W1.shape[1]
    C = W2.shape[1]
    Dp = 8
    NP = ((N + 127) // 128) * 128
    W = NC * NS
    epw = E // W
    NCH = (epw + K - 1) // K
    pad = NCH * K - epw

    src3 = jnp.pad(edge_index[0].reshape(W, epw), ((0, 0), (0, pad)),
                   constant_values=0).reshape(W, NCH, K)
    dst3 = jnp.pad(edge_index[1].reshape(W, epw), ((0, 0), (0, pad)),
                   constant_values=N).reshape(W, NCH, K)
    NCH2 = -(-NCH // NBUF) * NBUF
    pad2 = NCH2 * K - epw
    src3b = jnp.pad(edge_index[0].reshape(W, epw), ((0, 0), (0, pad2)),
                    constant_values=0).reshape(W, NCH2, K)
    padv = N + (jnp.arange(pad2, dtype=jnp.int32) % (NP - N))
    dst3b = jnp.concatenate(
        [edge_index[1].reshape(W, epw),
         jnp.broadcast_to(padv, (W, pad2))], axis=1).reshape(W, NCH2, K)

    xp = jnp.pad(x, ((0, NP - N), (0, 0)))
    ones8 = jnp.ones((K, 8), jnp.float32)
    zeros8 = jnp.zeros((NP, Dp), jnp.float32)
    zerosH = jnp.zeros((NP, H), jnp.float32)
    W2p = jnp.zeros((H, Dp), jnp.float32).at[:, :C].set(W2)
    b2p = jnp.zeros((1, Dp), jnp.float32).at[0, :C].set(b2)
    b1r = b1.reshape(1, H)

    parts = _build_deg_kernel(NCH, NP)(dst3, ones8, zeros8)
    g1 = pl.pallas_call(
        _tc1_body, out_shape=jax.ShapeDtypeStruct((NP, H), jnp.float32),
    )(parts, xp, W1)
    s1 = _build_scatter_kernel(NCH, NP, H)(src3, dst3, g1, zerosH)
    g2 = pl.pallas_call(
        _tc2_body, out_shape=jax.ShapeDtypeStruct((NP, Dp), jnp.float32),
    )(parts, s1, g1, b1r, W2p)
    s2 = _build_scatter_ring(NCH2, NP, Dp)(src3b, dst3b, g2, zeros8)
    out = pl.pallas_call(
        _tc3_body, out_shape=jax.ShapeDtypeStruct((NP, Dp), jnp.float32),
    )(parts, s2, g2, b2p)
    return out[:N, :C]


# speedup vs baseline: 1.2629x; 1.0309x over previous
import functools

import jax
import jax.numpy as jnp
from jax import lax
from jax.experimental import pallas as pl
from jax.experimental.pallas import tpu as pltpu
from jax.experimental.pallas import tpu_sc as plsc

NC = 2
NS = 16
K = 128
NBUF = 4


def _mesh():
    return plsc.VectorSubcoreMesh(core_axis_name="c", subcore_axis_name="s")


def _untiled():
    return pltpu.CompilerParams(use_tc_tiling_on_sc=False)


@functools.lru_cache(maxsize=None)
def _build_deg_kernel(NCH, N):
    rows = N // NS

    @functools.partial(
        pl.kernel,
        mesh=_mesh(),
        out_type=jax.ShapeDtypeStruct((NC, N, 8), jnp.float32),
        compiler_params=_untiled(),
        scratch_types=[
            pltpu.VMEM((NCH, K), jnp.int32),
            pltpu.VMEM((K, 8), jnp.float32),
            pltpu.VMEM_SHARED((N, 8), jnp.float32),
            pltpu.SemaphoreType.DMA,
        ],
    )
    def deg_kernel(dst_hbm, ones_hbm, zeros_hbm, out_hbm, didx, ones_v, acc_sh, sem):
        cid = lax.axis_index("c")
        sid = lax.axis_index("s")
        w = cid * NS + sid
        pltpu.sync_copy(zeros_hbm.at[pl.ds(sid * rows, rows)],
                        acc_sh.at[pl.ds(sid * rows, rows)])
        pltpu.sync_copy(dst_hbm.at[w], didx)
        pltpu.sync_copy(ones_hbm, ones_v)
        plsc.subcore_barrier()

        def fire(j, carry):
            pltpu.async_copy(ones_v, acc_sh.at[didx.at[j]], sem, add=True)
            return carry

        lax.fori_loop(0, NCH, fire, 0)

        def drain(j, carry):
            pltpu.make_async_copy(ones_v, acc_sh.at[didx.at[j]], sem).wait()
            return carry

        lax.fori_loop(0, NCH, drain, 0)
        plsc.subcore_barrier()
        pltpu.sync_copy(acc_sh.at[pl.ds(sid * rows, rows)],
                        out_hbm.at[cid, pl.ds(sid * rows, rows)])

    return deg_kernel


@functools.lru_cache(maxsize=None)
def _build_scatter_kernel(NCH, N, D):
    rows = N // NS
    half = (NCH - 1) // 2

    @functools.partial(
        pl.kernel,
        mesh=_mesh(),
        out_type=jax.ShapeDtypeStruct((NC, N, D), jnp.float32),
        compiler_params=_untiled(),
        scratch_types=[
            pltpu.VMEM((NCH, K), jnp.int32),
            pltpu.VMEM((NCH, K), jnp.int32),
            pltpu.VMEM((K, D), jnp.float32),
            pltpu.VMEM((K, D), jnp.float32),
            pltpu.SemaphoreType.DMA,
            pltpu.SemaphoreType.DMA,
            pltpu.VMEM_SHARED((N, D), jnp.float32),
        ],
    )
    def scat_kernel(src_hbm, dst_hbm, g_hbm, zeros_hbm, out_hbm,
                    sidx, didx, buf0, buf1, sem0, sem1, acc_sh):
        cid = lax.axis_index("c")
        sid = lax.axis_index("s")
        w = cid * NS + sid
        pltpu.sync_copy(zeros_hbm.at[pl.ds(sid * rows, rows)],
                        acc_sh.at[pl.ds(sid * rows, rows)])
        pltpu.sync_copy(src_hbm.at[w], sidx)
        pltpu.sync_copy(dst_hbm.at[w], didx)
        plsc.subcore_barrier()

        pltpu.async_copy(g_hbm.at[sidx.at[0]], buf0, sem0)

        def body(j, carry):
            c0 = 2 * j
            c1 = c0 + 1
            pltpu.make_async_copy(g_hbm.at[sidx.at[c0]], buf0, sem0).wait()
            pltpu.async_copy(g_hbm.at[sidx.at[c1]], buf1, sem1)
            pltpu.sync_copy(buf0, acc_sh.at[didx.at[c0]], add=True)
            pltpu.make_async_copy(g_hbm.at[sidx.at[c1]], buf1, sem1).wait()
            pltpu.async_copy(g_hbm.at[sidx.at[c1 + 1]], buf0, sem0)
            pltpu.sync_copy(buf1, acc_sh.at[didx.at[c1]], add=True)
            return carry

        lax.fori_loop(0, half, body, 0)
        pltpu.make_async_copy(g_hbm.at[sidx.at[NCH - 1]], buf0, sem0).wait()
        pltpu.sync_copy(buf0, acc_sh.at[didx.at[NCH - 1]], add=True)

        plsc.subcore_barrier()
        pltpu.sync_copy(acc_sh.at[pl.ds(sid * rows, rows)],
                        out_hbm.at[cid, pl.ds(sid * rows, rows)])

    return scat_kernel


def _build_scatter_ring(NCH, N, D, nbuf=NBUF):
    rows = N // NS
    assert NCH % nbuf == 0

    @functools.partial(
        pl.kernel,
        mesh=_mesh(),
        out_type=jax.ShapeDtypeStruct((NC, N, D), jnp.float32),
        compiler_params=_untiled(),
        scratch_types=[
            pltpu.VMEM((NCH, K), jnp.int32),
            pltpu.VMEM((NCH, K), jnp.int32),
            [pltpu.VMEM((K, D), jnp.float32) for _ in range(nbuf)],
            [pltpu.SemaphoreType.DMA for _ in range(nbuf)],
            [pltpu.SemaphoreType.DMA for _ in range(nbuf)],
            pltpu.VMEM_SHARED((N, D), jnp.float32),
        ],
    )
    def scat_kernel(src_hbm, dst_hbm, g_hbm, zeros_hbm, out_hbm,
                    sidx, didx, bufs, gsems, ssems, acc_sh):
        cid = lax.axis_index("c")
        sid = lax.axis_index("s")
        w = cid * NS + sid
        pltpu.sync_copy(zeros_hbm.at[pl.ds(sid * rows, rows)],
                        acc_sh.at[pl.ds(sid * rows, rows)])
        pltpu.sync_copy(src_hbm.at[w], sidx)
        pltpu.sync_copy(dst_hbm.at[w], didx)
        plsc.subcore_barrier()

        for b in range(nbuf - 1):
            pltpu.async_copy(g_hbm.at[sidx.at[b]], bufs[b], gsems[b])

        def body(jj, carry):
            for b in range(nbuf):
                c = nbuf * jj + b
                pltpu.make_async_copy(g_hbm.at[sidx.at[c]], bufs[b], gsems[b]).wait()
                pltpu.async_copy(bufs[b], acc_sh.at[didx.at[c]], ssems[b], add=True)
                nb = (b + nbuf - 1) % nbuf

                @pl.when(c + nbuf - 1 < NCH)
                def _():
                    @pl.when(c >= 1)
                    def _():
                        pltpu.make_async_copy(
                            bufs[nb], acc_sh.at[didx.at[jnp.maximum(c - 1, 0)]],
                            ssems[nb]).wait()
                    pltpu.async_copy(g_hbm.at[sidx.at[c + nbuf - 1]], bufs[nb],
                                     gsems[nb])
            return carry

        lax.fori_loop(0, NCH // nbuf, body, 0)
        for b in range(nbuf):
            c = NCH - nbuf + b
            pltpu.make_async_copy(bufs[c % nbuf], acc_sh.at[didx.at[c]],
                                  ssems[c % nbuf]).wait()

        plsc.subcore_barrier()
        pltpu.sync_copy(acc_sh.at[pl.ds(sid * rows, rows)],
                        out_hbm.at[cid, pl.ds(sid * rows, rows)])

    return scat_kernel


def _dis(parts):
    deg = parts[0, :, 0:1] + parts[1, :, 0:1] + 1.0
    return lax.rsqrt(deg)


def _tc1_body(parts_ref, x_ref, w1_ref, g1_ref):
    dis = _dis(parts_ref[...])
    h = jnp.dot(x_ref[...], w1_ref[...], preferred_element_type=jnp.float32,
                precision=lax.Precision.HIGHEST)
    g1_ref[...] = dis * h


def _tc2_body(parts_ref, s_ref, g1_ref, b1_ref, w2_ref, g2_ref):
    dis = _dis(parts_ref[...])
    s = s_ref[0] + s_ref[1] + g1_ref[...]
    h = jnp.maximum(dis * s + b1_ref[...], 0.0)
    g2_ref[...] = dis * jnp.dot(h, w2_ref[...], preferred_element_type=jnp.float32,
                                precision=lax.Precision.HIGHEST)


def _tc3_body(parts_ref, s_ref, g2_ref, b2_ref, out_ref):
    dis = _dis(parts_ref[...])
    out_ref[...] = dis * (s_ref[0] + s_ref[1] + g2_ref[...]) + b2_ref[...]


def kernel(x, edge_index, W1, b1, W2, b2):
    N, F = x.shape
    E = edge_index.shape[1]
    H = W1.shape[1]
    C = W2.shape[1]
    Dp = 8
    NP = ((N + 127) // 128) * 128
    W = NC * NS
    epw = E // W
    NCH = (epw + K - 1) // K
    pad = NCH * K - epw

    src3 = jnp.pad(edge_index[0].reshape(W, epw), ((0, 0), (0, pad)),
                   constant_values=0).reshape(W, NCH, K)
    dst3 = jnp.pad(edge_index[1].reshape(W, epw), ((0, 0), (0, pad)),
                   constant_values=N).reshape(W, NCH, K)
    NCH2 = -(-NCH // NBUF) * NBUF
    pad2 = NCH2 * K - epw
    src3b = jnp.pad(edge_index[0].reshape(W, epw), ((0, 0), (0, pad2)),
                    constant_values=0).reshape(W, NCH2, K)
    padv = N + (jnp.arange(pad2, dtype=jnp.int32) % (NP - N))
    dst3b = jnp.concatenate(
        [edge_index[1].reshape(W, epw),
         jnp.broadcast_to(padv, (W, pad2))], axis=1).reshape(W, NCH2, K)

    xp = jnp.pad(x, ((0, NP - N), (0, 0)))
    ones8 = jnp.ones((K, 8), jnp.float32)
    zeros8 = jnp.zeros((NP, Dp), jnp.float32)
    zerosH = jnp.zeros((NP, H), jnp.float32)
    W2p = jnp.zeros((H, Dp), jnp.float32).at[:, :C].set(W2)
    b2p = jnp.zeros((1, Dp), jnp.float32).at[0, :C].set(b2)
    b1r = b1.reshape(1, H)

    parts = _build_deg_kernel(NCH, NP)(dst3, ones8, zeros8)
    g1 = pl.pallas_call(
        _tc1_body, out_shape=jax.ShapeDtypeStruct((NP, H), jnp.float32),
    )(parts, xp, W1)
    s1 = _build_scatter_kernel(NCH, NP, H)(src3, dst3, g1, zerosH)
    g2 = pl.pallas_call(
        _tc2_body, out_shape=jax.ShapeDtypeStruct((NP, Dp), jnp.float32),
    )(parts, s1, g1, b1r, W2p)
    s2 = _build_scatter_ring(NCH2, NP, Dp, nbuf=8)(src3b, dst3b, g2, zeros8)
    out = pl.pallas_call(
        _tc3_body, out_shape=jax.ShapeDtypeStruct((NP, Dp), jnp.float32),
    )(parts, s2, g2, b2p)
    return out[:N, :C]
